# baseline (speedup 1.0000x reference)
import functools

import jax
import jax.numpy as jnp
from jax.experimental import pallas as pl
from jax.experimental.pallas import tpu as pltpu

_K = 8
_DS = 4


def _router_block(x_ref, w_ref, idx_ref, wgt_ref, logits_ref):
    k = pl.program_id(1)
    x_blk = x_ref[...]
    w = w_ref[...]
    partial = jax.lax.dot_general(
        w, x_blk, (((1,), (1,)), ((), ())),
        preferred_element_type=jnp.float32)

    @pl.when(k == 0)
    def _():
        logits_ref[...] = partial

    @pl.when(k > 0)
    def _():
        logits_ref[...] += partial

    @pl.when(k == _DS - 1)
    def _():
        lt = logits_ref[...]
        e, blk = lt.shape
        row = jax.lax.broadcasted_iota(jnp.int32, (e, blk), 0)
        work = lt
        vals = []
        idxs = []
        for _ in range(_K):
            m = jnp.max(work, axis=0, keepdims=True)
            idx = jnp.min(jnp.where(work == m, row, e), axis=0, keepdims=True)
            vals.append(m)
            idxs.append(idx)
            work = jnp.where(row == idx, -jnp.inf, work)
        topv = jnp.concatenate(vals, axis=0)
        topi = jnp.concatenate(idxs, axis=0)

        ex = jnp.exp(topv - topv[:1])
        wgt = ex / jnp.sum(ex, axis=0, keepdims=True)

        idx_ref[...] = topi
        wgt_ref[...] = wgt


@functools.partial(jax.jit, static_argnames=())
def kernel(x, W):
    b, s, d = x.shape
    e = W.shape[0]
    n = b * s
    blk = 1024
    xf = x.reshape(n, d)

    grid = (n // blk, _DS)
    idx_t, wgt_t, logits_t = pl.pallas_call(
        _router_block,
        grid=grid,
        in_specs=[
            pl.BlockSpec((blk, d // _DS), lambda i, k: (i, k)),
            pl.BlockSpec((e, d // _DS), lambda i, k: (0, k)),
        ],
        out_specs=[
            pl.BlockSpec((_K, blk), lambda i, k: (0, i)),
            pl.BlockSpec((_K, blk), lambda i, k: (0, i)),
            pl.BlockSpec((e, blk), lambda i, k: (0, i)),
        ],
        out_shape=[
            jax.ShapeDtypeStruct((_K, n), jnp.int32),
            jax.ShapeDtypeStruct((_K, n), jnp.float32),
            jax.ShapeDtypeStruct((e, n), jnp.float32),
        ],
        compiler_params=pltpu.CompilerParams(
            dimension_semantics=("parallel", "arbitrary"),
        ),
    )(xf, W)

    return (idx_t.T.reshape(b, s, _K), wgt_t.T.reshape(b, s, _K),
            logits_t.T.reshape(b, s, e))


# speedup vs baseline: 1.3510x; 1.3510x over previous
import functools

import jax
import jax.numpy as jnp
from jax.experimental import pallas as pl
from jax.experimental.pallas import tpu as pltpu

_K = 8


def _router_block(x_ref, w_ref, idx_ref, wgt_ref, logits_ref):
    x_blk = x_ref[...]
    w = w_ref[...]
    lt = jax.lax.dot_general(
        w, x_blk, (((1,), (1,)), ((), ())),
        preferred_element_type=jnp.float32)
    logits_ref[...] = lt

    e, blk = lt.shape
    row = jax.lax.broadcasted_iota(jnp.int32, (e, blk), 0)
    work = lt
    vals = []
    idxs = []
    for _ in range(_K):
        m = jnp.max(work, axis=0, keepdims=True)
        idx = jnp.min(jnp.where(work == m, row, e), axis=0, keepdims=True)
        vals.append(m)
        idxs.append(idx)
        work = jnp.where(row == idx, -jnp.inf, work)
    topv = jnp.concatenate(vals, axis=0)
    topi = jnp.concatenate(idxs, axis=0)

    ex = jnp.exp(topv - topv[:1])
    wgt = ex / jnp.sum(ex, axis=0, keepdims=True)

    idx_ref[...] = topi
    wgt_ref[...] = wgt


@functools.partial(jax.jit, static_argnames=())
def kernel(x, W):
    b, s, d = x.shape
    e = W.shape[0]
    n = b * s
    blk = 1024
    xf = x.reshape(n, d)

    grid = (n // blk,)
    idx_t, wgt_t, logits_t = pl.pallas_call(
        _router_block,
        grid=grid,
        in_specs=[
            pl.BlockSpec((blk, d), lambda i: (i, 0)),
            pl.BlockSpec((e, d), lambda i: (0, 0)),
        ],
        out_specs=[
            pl.BlockSpec((_K, blk), lambda i: (0, i)),
            pl.BlockSpec((_K, blk), lambda i: (0, i)),
            pl.BlockSpec((e, blk), lambda i: (0, i)),
        ],
        out_shape=[
            jax.ShapeDtypeStruct((_K, n), jnp.int32),
            jax.ShapeDtypeStruct((_K, n), jnp.float32),
            jax.ShapeDtypeStruct((e, n), jnp.float32),
        ],
        compiler_params=pltpu.CompilerParams(
            dimension_semantics=("parallel",),
        ),
    )(xf, W)

    return (idx_t.T.reshape(b, s, _K), wgt_t.T.reshape(b, s, _K),
            logits_t.T.reshape(b, s, e))
